# Initial kernel scaffold; baseline (speedup 1.0000x reference)
import jax
import jax.numpy as jnp
from jax.experimental import pallas as pl


def kernel(x, edge_index, edge_mask, W, b, gamma, beta, prelu_a):
    raise NotImplementedError("write your pallas kernel here")


# speedup vs baseline: 12.1143x; 12.1143x over previous
import functools

import jax
import jax.numpy as jnp
from jax import lax
from jax.experimental import pallas as pl
from jax.experimental.pallas import tpu as pltpu
from jax.experimental.pallas import tpu_sc as plsc

B = 4
N = 10000
K = 16
D = 128
NS = 16
NBATCH_PER_CORE = 2
NPAD = 10112
SUB_ROWS = NPAD // NS
CHUNK = 128
NCHUNK = 5
SPAD = 10240
ACC_ROWS = NPAD
STRIP = ACC_ROWS // NS
GARBAGE = N
MM_BLK = 128
NBLK = NPAD // MM_BLK
LN_BLK = 80
NLBLK = N // LN_BLK
CNT = float(B * N * D)

_mesh = plsc.VectorSubcoreMesh(core_axis_name="c", subcore_axis_name="s")


@functools.partial(
    pl.kernel,
    out_type=jax.ShapeDtypeStruct((B, ACC_ROWS, D), jnp.float32),
    mesh=_mesh,
    scratch_types=[
        pltpu.VMEM((CHUNK, D), jnp.float32),
        pltpu.VMEM((K, NCHUNK, CHUNK), jnp.int32),
        pltpu.VMEM_SHARED((ACC_ROWS, D), jnp.float32),
    ],
)
def _sc_degree(idx_hbm, ones_hbm, zeros_hbm, deg_hbm, ones_v, idx_v, acc):
    c = lax.axis_index("c")
    sid = lax.axis_index("s")
    pltpu.sync_copy(ones_hbm, ones_v)
    for bi in range(NBATCH_PER_CORE):
        b = c + 2 * bi
        pltpu.sync_copy(zeros_hbm, acc.at[pl.ds(sid * STRIP, STRIP)])
        plsc.subcore_barrier()
        pltpu.sync_copy(idx_hbm.at[b, sid], idx_v)
        for k in range(K):
            for m in range(NCHUNK):
                pltpu.sync_copy(ones_v, acc.at[idx_v.at[k, m]], add=True)
        plsc.subcore_barrier()
        pltpu.sync_copy(acc.at[pl.ds(sid * STRIP, STRIP)],
                        deg_hbm.at[b, pl.ds(sid * STRIP, STRIP)])
        plsc.subcore_barrier()


@functools.partial(
    pl.kernel,
    out_type=jax.ShapeDtypeStruct((B, ACC_ROWS, D), jnp.float32),
    mesh=_mesh,
    scratch_types=[
        pltpu.VMEM((CHUNK, D), jnp.float32),
        pltpu.VMEM((K, NCHUNK, CHUNK), jnp.int32),
        pltpu.VMEM_SHARED((ACC_ROWS, D), jnp.float32),
    ],
)
def _sc_scatter(s_hbm, idx_hbm, zeros_hbm, out_hbm, src_v, idx_v, acc):
    c = lax.axis_index("c")
    sid = lax.axis_index("s")
    for bi in range(NBATCH_PER_CORE):
        b = c + 2 * bi
        pltpu.sync_copy(zeros_hbm, acc.at[pl.ds(sid * STRIP, STRIP)])
        plsc.subcore_barrier()
        pltpu.sync_copy(idx_hbm.at[b, sid], idx_v)
        for m in range(NCHUNK):
            pltpu.sync_copy(
                s_hbm.at[b, pl.ds(sid * SUB_ROWS + m * CHUNK, CHUNK)], src_v)
            for k in range(K):
                pltpu.sync_copy(src_v, acc.at[idx_v.at[k, m]], add=True)
        plsc.subcore_barrier()
        pltpu.sync_copy(acc.at[pl.ds(sid * STRIP, STRIP)],
                        out_hbm.at[b, pl.ds(sid * STRIP, STRIP)])
        plsc.subcore_barrier()


def _mm_body(x_ref, w_ref, deg_ref, s_ref):
    dinv = lax.rsqrt(deg_ref[0, :, 0] + 1.0)
    h = jnp.dot(x_ref[0], w_ref[:], preferred_element_type=jnp.float32)
    s_ref[0] = h * dinv[:, None]


def _c1_body(acc_ref, s_ref, deg_ref, b_ref, y_ref, p_ref):
    i = pl.program_id(1)
    dinv = lax.rsqrt(deg_ref[0, :, 0] + 1.0)
    y = dinv[:, None] * (acc_ref[0] + s_ref[0]) + b_ref[:][None, :]
    y_ref[0] = y
    row = i * MM_BLK + lax.broadcasted_iota(jnp.int32, (MM_BLK, 1), 0)
    ym = jnp.where(row < N, y, 0.0)
    lane = lax.broadcasted_iota(jnp.int32, (1, 2), 1)
    p_ref[0] = jnp.where(lane == 0, jnp.sum(ym), jnp.sum(ym * ym))


def _c2_body(y_ref, p_ref, g_ref, beta_ref, a_ref, o_ref):
    tot = jnp.sum(p_ref[:, 0, :], axis=0)
    mu = tot[0] / CNT
    var = tot[1] / CNT - mu * mu
    inv = lax.rsqrt(var + 1e-5)
    o = (y_ref[0] - mu) * inv * g_ref[:][None, :] + beta_ref[:][None, :]
    a = a_ref[0]
    o_ref[:] = jnp.where(o >= 0.0, o, a * o)


def _build_dst(edge_index, edge_mask):
    ei = edge_index[:, :, 1:].astype(jnp.int32)
    em = edge_mask[:, :, 1:]
    dst = jnp.where(em, ei, GARBAGE)
    dst = jnp.pad(dst, ((0, 0), (0, NPAD - N), (0, 0)),
                  constant_values=GARBAGE)
    dst = dst.reshape(B, NS, SUB_ROWS, K)
    dst = jnp.pad(dst, ((0, 0), (0, 0), (0, NCHUNK * CHUNK - SUB_ROWS), (0, 0)),
                  constant_values=GARBAGE)
    return jnp.transpose(dst, (0, 1, 3, 2)).reshape(B, NS, K, NCHUNK, CHUNK)


def kernel(x, edge_index, edge_mask, W, b, gamma, beta, prelu_a):
    x4 = x[:, 0, :].reshape(B, N, D)
    dst = _build_dst(edge_index, edge_mask)

    onesd = jnp.ones((CHUNK, D), jnp.float32)
    zerosd = jnp.zeros((STRIP, D), jnp.float32)

    deg8 = _sc_degree(dst, onesd, zerosd)

    s_pad = pl.pallas_call(
        _mm_body,
        grid=(B, NBLK),
        in_specs=[
            pl.BlockSpec((1, MM_BLK, D), lambda bb, i: (bb, i, 0)),
            pl.BlockSpec((D, D), lambda bb, i: (0, 0)),
            pl.BlockSpec((1, MM_BLK, D), lambda bb, i: (bb, i, 0)),
        ],
        out_specs=pl.BlockSpec((1, MM_BLK, D), lambda bb, i: (bb, i, 0)),
        out_shape=jax.ShapeDtypeStruct((B, SPAD, D), jnp.float32),
    )(x4, W, deg8)

    acc = _sc_scatter(s_pad, dst, zerosd)

    y_pre, partials = pl.pallas_call(
        _c1_body,
        grid=(B, NBLK),
        in_specs=[
            pl.BlockSpec((1, MM_BLK, D), lambda bb, i: (bb, i, 0)),
            pl.BlockSpec((1, MM_BLK, D), lambda bb, i: (bb, i, 0)),
            pl.BlockSpec((1, MM_BLK, D), lambda bb, i: (bb, i, 0)),
            pl.BlockSpec((D,), lambda bb, i: (0,)),
        ],
        out_specs=[
            pl.BlockSpec((1, MM_BLK, D), lambda bb, i: (bb, i, 0)),
            pl.BlockSpec((1, 1, 2), lambda bb, i: (bb * NBLK + i, 0, 0)),
        ],
        out_shape=[
            jax.ShapeDtypeStruct((B, NPAD, D), jnp.float32),
            jax.ShapeDtypeStruct((B * NBLK, 1, 2), jnp.float32),
        ],
    )(acc, s_pad, deg8, b)

    out = pl.pallas_call(
        _c2_body,
        grid=(B, NLBLK),
        in_specs=[
            pl.BlockSpec((1, LN_BLK, D), lambda bb, i: (bb, i, 0)),
            pl.BlockSpec((B * NBLK, 1, 2), lambda bb, i: (0, 0, 0)),
            pl.BlockSpec((D,), lambda bb, i: (0,)),
            pl.BlockSpec((D,), lambda bb, i: (0,)),
            pl.BlockSpec((1,), lambda bb, i: (0,)),
        ],
        out_specs=pl.BlockSpec((LN_BLK, D), lambda bb, i: (bb * NLBLK + i, 0)),
        out_shape=jax.ShapeDtypeStruct((B * N, D), jnp.float32),
    )(y_pre, partials, gamma, beta, prelu_a)
    return out
